# baseline (speedup 1.0000x reference)
import functools

import jax
import jax.numpy as jnp
from jax import lax
from jax.experimental import pallas as pl
from jax.experimental.pallas import tpu as pltpu
from jax.experimental.pallas import tpu_sc as plsc

B = 4096
DIM = 32
NN = 8
NREL = 32
ROWS_PER_B = 1 + NN + NN * NN

NC, NS = 2, 16
NW = NC * NS
BPW = B // NW
CH = 128
NCHUNK = ROWS_PER_B * BPW // CH

BB = 256


def _sc_gather(ent_embed, usr_embed, i0, i1, i2, uids):
    mesh = plsc.VectorSubcoreMesh(
        core_axis_name="c", subcore_axis_name="s", num_cores=NC, num_subcores=NS
    )

    @functools.partial(
        pl.kernel,
        out_type=(
            jax.ShapeDtypeStruct((NW * NN * NN, CH, DIM), jnp.float32),
            jax.ShapeDtypeStruct((NW * NN, CH, DIM), jnp.float32),
            jax.ShapeDtypeStruct((NW, CH, DIM), jnp.float32),
            jax.ShapeDtypeStruct((NW, CH, DIM), jnp.float32),
        ),
        mesh=mesh,
        compiler_params=pltpu.CompilerParams(use_tc_tiling_on_sc=False),
        scratch_types=[
            pltpu.VMEM((NN * NN, CH), jnp.int32),
            pltpu.VMEM((NN, CH), jnp.int32),
            pltpu.VMEM((CH,), jnp.int32),
            pltpu.VMEM((CH,), jnp.int32),
            pltpu.VMEM((8, CH, DIM), jnp.float32),
            pltpu.VMEM((NN, CH, DIM), jnp.float32),
            pltpu.VMEM((CH, DIM), jnp.float32),
            pltpu.VMEM((CH, DIM), jnp.float32),
            pltpu.SemaphoreType.DMA,
            pltpu.SemaphoreType.DMA,
            pltpu.SemaphoreType.DMA,
            pltpu.SemaphoreType.DMA,
            pltpu.SemaphoreType.DMA,
            pltpu.SemaphoreType.DMA,
        ],
    )
    def k(ent_hbm, usr_hbm, i0_hbm, i1_hbm, i2_hbm, uid_hbm,
          e2_hbm, e1_hbm, e0_hbm, u_hbm,
          i2_v, i1_v, i0_v, uid_v, bufs, e1b, e0b, ub,
          gsem0, gsem1, wsem0, wsem1, esem, wesem):
        wid = lax.axis_index("s") * NC + lax.axis_index("c")
        row0 = wid * BPW
        pltpu.sync_copy(i2_hbm.at[wid], i2_v)
        pltpu.sync_copy(i1_hbm.at[wid], i1_v)
        pltpu.sync_copy(i0_hbm.at[wid], i0_v)
        pltpu.sync_copy(uid_hbm.at[wid], uid_v)

        for c in range(NN):
            pltpu.async_copy(ent_hbm.at[i1_v.at[c]], e1b.at[c], esem)
        pltpu.async_copy(ent_hbm.at[i0_v], e0b, esem)
        pltpu.async_copy(usr_hbm.at[uid_v], ub, esem)

        def g_start(j, slot, sem):
            pltpu.async_copy(ent_hbm.at[i2_v.at[j]], bufs.at[slot], sem)

        def g_wait(j, slot, sem):
            pltpu.make_async_copy(ent_hbm.at[i2_v.at[j]], bufs.at[slot], sem).wait()

        def w_start(j, slot, sem):
            pltpu.async_copy(bufs.at[slot], e2_hbm.at[wid * NN * NN + j], sem)

        def w_wait(j, slot, sem):
            pltpu.make_async_copy(bufs.at[slot],
                                  e2_hbm.at[wid * NN * NN + j], sem).wait()

        for c in range(4):
            g_start(c, c, gsem0)
        for c in range(4):
            g_start(4 + c, 4 + c, gsem1)

        def body(i, carry):
            j0 = 8 * i
            for c in range(4):
                g_wait(j0 + c, c, gsem0)
            for c in range(4):
                w_start(j0 + c, c, wsem0)
            for c in range(4):
                g_wait(j0 + 4 + c, 4 + c, gsem1)
            for c in range(4):
                w_start(j0 + 4 + c, 4 + c, wsem1)
            for c in range(4):
                w_wait(j0 + c, c, wsem0)
            for c in range(4):
                g_start(j0 + 8 + c, c, gsem0)
            for c in range(4):
                w_wait(j0 + 4 + c, 4 + c, wsem1)
            for c in range(4):
                g_start(j0 + 12 + c, 4 + c, gsem1)
            return carry

        lax.fori_loop(0, 7, body, 0)
        for c in range(4):
            g_wait(56 + c, c, gsem0)
        for c in range(4):
            w_start(56 + c, c, wsem0)
        for c in range(4):
            g_wait(60 + c, 4 + c, gsem1)
        for c in range(4):
            w_start(60 + c, 4 + c, wsem1)
        for c in range(4):
            w_wait(56 + c, c, wsem0)
        for c in range(4):
            w_wait(60 + c, 4 + c, wsem1)

        for c in range(NN):
            pltpu.make_async_copy(ent_hbm.at[i1_v.at[c]], e1b.at[c], esem).wait()
        pltpu.make_async_copy(ent_hbm.at[i0_v], e0b, esem).wait()
        pltpu.make_async_copy(usr_hbm.at[uid_v], ub, esem).wait()
        for c in range(NN):
            pltpu.async_copy(e1b.at[c], e1_hbm.at[wid * NN + c], wesem)
        pltpu.async_copy(e0b, e0_hbm.at[wid], wesem)
        pltpu.async_copy(ub, u_hbm.at[wid], wesem)
        for c in range(NN):
            pltpu.make_async_copy(e1b.at[c], e1_hbm.at[wid * NN + c], wesem).wait()
        pltpu.make_async_copy(e0b, e0_hbm.at[wid], wesem).wait()
        pltpu.make_async_copy(ub, u_hbm.at[wid], wesem).wait()

    return k(ent_embed, usr_embed, i0, i1, i2, uids)


NPOS = NN + NN * NN
NG = 1 + NN
QW = NPOS * DIM
GW = NG * DIM


def _np_consts():
    import numpy as np
    eye32 = np.eye(DIM, dtype=np.float32)
    t32 = np.tile(eye32, (1, NPOS))
    r72 = np.repeat(np.eye(NPOS, dtype=np.float32), DIM, axis=1)
    c72 = r72.T.copy()
    d9 = np.repeat(np.eye(NG, dtype=np.float32), NN, axis=0)
    e9 = np.repeat(np.eye(NG, dtype=np.float32), DIM, axis=1)
    h = np.kron(d9, eye32)
    r8 = np.repeat(np.eye(NN, dtype=np.float32), DIM, axis=1)
    hs = np.tile(eye32, (NN, 1))
    tb = np.tile(eye32, (1, NG))
    tbt = tb.T.copy()
    bd = np.kron(np.eye(NG, dtype=np.float32), np.ones((DIM, DIM), np.float32))
    return t32, r72, c72, d9, e9, h, r8, hs, tb, tbt, bd


def _dot(x, y):
    return lax.dot_general(x, y, (((1,), (0,)), ((), ())),
                           preferred_element_type=jnp.float32)


def _mm(x, w):
    return lax.dot_general(x, w, (((1,), (1,)), ((), ())),
                           preferred_element_type=jnp.float32)


def _tc_body(U_ref, E0_ref, E1_ref, E2_ref, ids_ref, rel_ref, W_ref, b_ref,
             t32_ref, r72_ref, c72_ref, d9_ref, e9_ref, h_ref, r8_ref, hs_ref,
             tb_ref, tbt_ref, bd_ref, out_ref):
    U = U_ref[...]
    E0 = E0_ref[...]
    E1 = E1_ref[...]
    E2 = E2_ref[...]
    idsf = ids_ref[...]
    rel = rel_ref[...]
    W = W_ref[...]
    bvec = b_ref[...]

    S = _mm(U, rel)
    mx = jnp.max(S, axis=1, keepdims=True)
    expS = jnp.exp(S - mx)

    eh = _dot(expS, t32_ref[...])
    idr = _dot(idsf, r72_ref[...])
    lmod = lax.rem(lax.broadcasted_iota(jnp.int32, (BB, QW), 1), DIM)
    eflat = jnp.where(idr.astype(jnp.int32) == lmod, eh, 0.0)

    e = _dot(eflat, c72_ref[...])
    den = _dot(e, d9_ref[...])
    denr = _dot(den, e9_ref[...])
    wrep = _dot(e, r72_ref[...])

    En = jnp.concatenate([E1, E2], axis=1)
    agg = _dot(wrep * En, h_ref[...])
    sv = jnp.concatenate([E0, E1], axis=1)
    pre = sv + agg / denr

    wt = _mm(tbt_ref[...], W)
    bw = _dot(wt, tb_ref[...]) * bd_ref[...]
    btile = _dot(bvec, tb_ref[...])
    o = jax.nn.sigmoid(_dot(pre, bw) + btile)

    o0 = o[:, :DIM]
    o1 = o[:, DIM:]
    p0 = e[:, :NN] / den[:, 0:1]
    w0 = _dot(p0, r8_ref[...])
    aggf = _dot(w0 * o1, hs_ref[...])
    fin = jnp.tanh(_mm(o0 + aggf, W) + bvec)
    out_ref[...] = jax.nn.sigmoid(jnp.sum(U * fin, axis=1))


def _tc_dense(U, E0, E1, E2, idsf, rel, W, bvec, interpret=False):
    consts = _np_consts()
    grid = (B // BB,)
    bcast = lambda shape: pl.BlockSpec(shape, lambda i: tuple(0 for _ in shape))
    blk = lambda shape: pl.BlockSpec(shape, lambda i: (i,) + tuple(0 for _ in shape[1:]))
    return pl.pallas_call(
        _tc_body,
        grid=grid,
        in_specs=[
            blk((BB, DIM)),
            blk((BB, DIM)),
            blk((BB, NN * DIM)),
            blk((BB, NN * NN * DIM)),
            blk((BB, NPOS)),
            bcast((NREL, DIM)),
            bcast((DIM, DIM)),
            bcast((1, DIM)),
        ] + [bcast(c.shape) for c in consts],
        out_specs=pl.BlockSpec((BB,), lambda i: (i,)),
        out_shape=jax.ShapeDtypeStruct((B,), jnp.float32),
        interpret=interpret,
    )(U, E0, E1, E2, idsf, rel, W, bvec, *consts)


def kernel(usr_id, usr_embed, ent_id_0, ent_id_1, ent_id_2, ent_embed,
           rel_id_0, rel_id_1, rel_embed, W, b):
    i0 = ent_id_0.reshape(NW, CH)
    i1 = ent_id_1.reshape(NW, NN, CH)
    i2 = ent_id_2.reshape(NW, NN * NN, CH)
    uids = usr_id.reshape(NW, CH)
    E2o, E1o, E0o, Uo = _sc_gather(ent_embed, usr_embed, i0, i1, i2, uids)
    E2 = E2o.reshape(B, NN * NN * DIM)
    E1 = E1o.reshape(B, NN * DIM)
    E0 = E0o.reshape(B, DIM)
    U = Uo.reshape(B, DIM)
    idsf = jnp.concatenate([rel_id_0, rel_id_1], axis=1).astype(jnp.float32)
    return _tc_dense(U, E0, E1, E2, idsf, rel_embed, W, b.reshape(1, DIM))
